# Initial kernel scaffold; baseline (speedup 1.0000x reference)
import jax
import jax.numpy as jnp
from jax.experimental import pallas as pl


def kernel(x, edge_index, W1, b1, Wg, att_src, att_dst, bg, W2, b2):
    raise NotImplementedError("write your pallas kernel here")


# speedup vs baseline: 1.2482x; 1.2482x over previous
import functools

import jax
import jax.numpy as jnp
from jax.experimental import pallas as pl

_N = 100000
_HID = 32
_HEADS = 2


def _relu_kernel(x_ref, o_ref):
    o_ref[...] = jnp.maximum(x_ref[...], 0.0)


def _pallas_relu(x):
    rows = x.shape[0]
    blk = 10000
    return pl.pallas_call(
        _relu_kernel,
        grid=(rows // blk,),
        in_specs=[pl.BlockSpec((blk, x.shape[1]), lambda i: (i, 0))],
        out_specs=pl.BlockSpec((blk, x.shape[1]), lambda i: (i, 0)),
        out_shape=jax.ShapeDtypeStruct(x.shape, x.dtype),
    )(x)


def kernel(x, edge_index, W1, b1, Wg, att_src, att_dst, bg, W2, b2):
    n = x.shape[0]
    src = edge_index[0]
    dst = edge_index[1]

    deg = jnp.zeros((n,), jnp.float32).at[dst].add(1.0) + 1.0
    dis = deg ** -0.5

    h1 = x @ W1.T
    hs1 = h1 * dis[:, None]
    agg1 = jnp.zeros((n, _HID), jnp.float32).at[dst].add(hs1[src])
    out1 = dis[:, None] * (agg1 + hs1) + b1
    r1 = _pallas_relu(out1)

    h2 = (r1 @ Wg.T).reshape(n, _HEADS, _HID)
    a_s = (h2 * att_src).sum(-1)
    a_d = (h2 * att_dst).sum(-1)
    alpha = a_s[src] + a_d[dst]
    alpha = jnp.where(alpha > 0, alpha, 0.2 * alpha)
    ex = jnp.exp(alpha)
    a_self = a_s + a_d
    ex_self = jnp.exp(jnp.where(a_self > 0, a_self, 0.2 * a_self))
    den = jnp.zeros((n, _HEADS), jnp.float32).at[dst].add(ex) + ex_self
    msum = jnp.zeros((n, _HEADS, _HID), jnp.float32).at[dst].add(
        h2[src] * ex[:, :, None]) + h2 * ex_self[:, :, None]
    out2 = (msum / (den[:, :, None] + 1e-16)).mean(axis=1) + bg
    r2 = _pallas_relu(out2)

    h3 = (r2 @ W2.T)[:, 0]
    hs3 = h3 * dis
    agg3 = jnp.zeros((n,), jnp.float32).at[dst].add(hs3[src])
    out3 = dis * (agg3 + hs3) + b2[0]
    return out3


# speedup vs baseline: 1.2943x; 1.0369x over previous
import functools

import jax
import jax.numpy as jnp
from jax import lax
from jax.experimental import pallas as pl
from jax.experimental.pallas import tpu as pltpu
from jax.experimental.pallas import tpu_sc as plsc

_N = 100000
_HID = 32
_HEADS = 2
_E = 1600000

_NC = 2
_NS = 16
_NW = _NC * _NS
_ROWS = 784
_NPAD = _ROWS * 128


def _zero_rows128(ref, n):
    z = jnp.zeros((16,), jnp.float32)

    def body(i, carry):
        for j in range(8):
            ref[i, pl.ds(j * 16, 16)] = z
        return carry

    lax.fori_loop(0, n, body, 0, unroll=2)


def _count_body(dst_hbm, out_hbm, acc, dvb):
    c = lax.axis_index("c")
    s = lax.axis_index("s")
    epb = _E // _NW
    base = (c * _NS + s) * epb
    _zero_rows128(acc, _ROWS)
    ones = jnp.full((16,), 1.0, jnp.float32)

    def chunk(k, carry):
        pltpu.sync_copy(dst_hbm.at[pl.ds(base + k * 2000, 2000)], dvb)

        def inner(j, c2):
            dvv = dvb[pl.ds(j * 16, 16)]
            plsc.addupdate_scatter(acc, [dvv >> 7, dvv & 127], ones)
            return c2

        lax.fori_loop(0, 125, inner, 0, unroll=4)
        return carry

    lax.fori_loop(0, epb // 2000, chunk, 0)
    pltpu.sync_copy(acc, out_hbm.at[c, s])


_NPS = 100096
_CB = 1000


def _spmm16_body(tabs_hbm, sv_hbm, dv_hbm, zero_hbm, out_hbm, svb, dvb, rows, spacc, sem):
    c = lax.axis_index("c")
    s = lax.axis_index("s")
    epb = _E // _NS
    base = s * epb
    tab = tabs_hbm.at[c]

    pltpu.sync_copy(zero_hbm.at[pl.ds(s * 6256, 6256)], spacc.at[pl.ds(s * 6256, 6256)])
    plsc.subcore_barrier()

    def chunk(k, carry):
        pltpu.sync_copy(sv_hbm.at[pl.ds(base + k * _CB, _CB)], svb)
        pltpu.sync_copy(dv_hbm.at[pl.ds(base + k * _CB, _CB)], dvb)
        pltpu.async_copy(tab.at[svb], rows, sem).wait()
        pltpu.sync_copy(rows, spacc.at[dvb], add=True)
        return carry

    lax.fori_loop(0, epb // _CB, chunk, 0)
    plsc.subcore_barrier()
    pltpu.sync_copy(spacc.at[pl.ds(s * 6256, 6256)], out_hbm.at[c, pl.ds(s * 6256, 6256)])


_spmm16_kernel = pl.kernel(
    _spmm16_body,
    out_type=jax.ShapeDtypeStruct((_NC, _NPS, 16), jnp.float32),
    mesh=plsc.VectorSubcoreMesh(core_axis_name="c", subcore_axis_name="s"),
    compiler_params=pltpu.CompilerParams(needs_layout_passes=False, use_tc_tiling_on_sc=False),
    scratch_types=[
        pltpu.VMEM((_CB,), jnp.int32),
        pltpu.VMEM((_CB,), jnp.int32),
        pltpu.VMEM((_CB, 16), jnp.float32),
        pltpu.VMEM_SHARED((_NPS, 16), jnp.float32),
        pltpu.SemaphoreType.DMA,
    ],
)


_count_kernel = pl.kernel(
    _count_body,
    out_type=jax.ShapeDtypeStruct((_NC, _NS, _ROWS, 128), jnp.float32),
    mesh=plsc.VectorSubcoreMesh(core_axis_name="c", subcore_axis_name="s"),
    compiler_params=pltpu.CompilerParams(needs_layout_passes=False),
    scratch_types=[
        pltpu.VMEM((_ROWS, 128), jnp.float32),
        pltpu.VMEM((2000,), jnp.int32),
    ],
)


def _relu_kernel(x_ref, o_ref):
    o_ref[...] = jnp.maximum(x_ref[...], 0.0)


def _pallas_relu(x):
    rows = x.shape[0]
    blk = 10000
    return pl.pallas_call(
        _relu_kernel,
        grid=(rows // blk,),
        in_specs=[pl.BlockSpec((blk, x.shape[1]), lambda i: (i, 0))],
        out_specs=pl.BlockSpec((blk, x.shape[1]), lambda i: (i, 0)),
        out_shape=jax.ShapeDtypeStruct(x.shape, x.dtype),
    )(x)


def kernel(x, edge_index, W1, b1, Wg, att_src, att_dst, bg, W2, b2):
    n = x.shape[0]
    src = edge_index[0]
    dst = edge_index[1]

    degp = _count_kernel(dst).reshape(_NW, _NPAD)[:, :n]
    deg = degp.sum(0) + 1.0
    dis = deg ** -0.5

    h1 = x @ W1.T
    hs1 = h1 * dis[:, None]
    hs1h = jnp.stack([hs1[:, :16], hs1[:, 16:]])
    zpad = jnp.zeros((_NPS, 16), jnp.float32)
    aggp = _spmm16_kernel(hs1h, src, dst, zpad)
    agg1 = jnp.concatenate([aggp[0, :n], aggp[1, :n]], axis=1)
    out1 = dis[:, None] * (agg1 + hs1) + b1
    r1 = _pallas_relu(out1)

    h2 = (r1 @ Wg.T).reshape(n, _HEADS, _HID)
    a_s = (h2 * att_src).sum(-1)
    a_d = (h2 * att_dst).sum(-1)
    alpha = a_s[src] + a_d[dst]
    alpha = jnp.where(alpha > 0, alpha, 0.2 * alpha)
    ex = jnp.exp(alpha)
    a_self = a_s + a_d
    ex_self = jnp.exp(jnp.where(a_self > 0, a_self, 0.2 * a_self))
    den = jnp.zeros((n, _HEADS), jnp.float32).at[dst].add(ex) + ex_self
    msum = jnp.zeros((n, _HEADS, _HID), jnp.float32).at[dst].add(
        h2[src] * ex[:, :, None]) + h2 * ex_self[:, :, None]
    out2 = (msum / (den[:, :, None] + 1e-16)).mean(axis=1) + bg
    r2 = _pallas_relu(out2)

    h3 = (r2 @ W2.T)[:, 0]
    hs3 = h3 * dis
    agg3 = jnp.zeros((n,), jnp.float32).at[dst].add(hs3[src])
    out3 = dis * (agg3 + hs3) + b2[0]
    return out3


# speedup vs baseline: 70.9706x; 54.8341x over previous
import functools

import jax
import jax.numpy as jnp
from jax import lax
from jax.experimental import pallas as pl
from jax.experimental.pallas import tpu as pltpu
from jax.experimental.pallas import tpu_sc as plsc

_N = 100000
_HID = 32
_HEADS = 2
_E = 1600000

_NC = 2
_NS = 16
_NW = _NC * _NS
_ROWS = 784
_NPAD = _ROWS * 128


def _zero_rows128(ref, n):
    z = jnp.zeros((16,), jnp.float32)

    def body(i, carry):
        for j in range(8):
            ref[i, pl.ds(j * 16, 16)] = z
        return carry

    lax.fori_loop(0, n, body, 0, unroll=2)


def _count_body(dst_hbm, out_hbm, acc, dvb):
    c = lax.axis_index("c")
    s = lax.axis_index("s")
    epb = _E // _NW
    base = (c * _NS + s) * epb
    _zero_rows128(acc, _ROWS)
    ones = jnp.full((16,), 1.0, jnp.float32)

    def chunk(k, carry):
        pltpu.sync_copy(dst_hbm.at[pl.ds(base + k * 2000, 2000)], dvb)

        def inner(j, c2):
            dvv = dvb[pl.ds(j * 16, 16)]
            plsc.addupdate_scatter(acc, [dvv >> 7, dvv & 127], ones)
            return c2

        lax.fori_loop(0, 125, inner, 0, unroll=4)
        return carry

    lax.fori_loop(0, epb // 2000, chunk, 0)
    pltpu.sync_copy(acc, out_hbm.at[c, s])


_NPS = 100096
_CB = 1000


def _spmm16_body(tabs_hbm, sv_hbm, dv_hbm, zero_hbm, out_hbm, svb, dvb, rows, spacc, sem):
    c = lax.axis_index("c")
    s = lax.axis_index("s")
    epb = _E // _NS
    base = s * epb
    tab = tabs_hbm.at[c]

    pltpu.sync_copy(zero_hbm.at[pl.ds(s * 6256, 6256)], spacc.at[pl.ds(s * 6256, 6256)])
    plsc.subcore_barrier()

    def chunk(k, carry):
        pltpu.sync_copy(sv_hbm.at[pl.ds(base + k * _CB, _CB)], svb)
        pltpu.sync_copy(dv_hbm.at[pl.ds(base + k * _CB, _CB)], dvb)
        pltpu.async_copy(tab.at[svb], rows, sem).wait()
        pltpu.sync_copy(rows, spacc.at[dvb], add=True)
        return carry

    lax.fori_loop(0, epb // _CB, chunk, 0)
    plsc.subcore_barrier()
    pltpu.sync_copy(spacc.at[pl.ds(s * 6256, 6256)], out_hbm.at[c, pl.ds(s * 6256, 6256)])


_spmm16_kernel = pl.kernel(
    _spmm16_body,
    out_type=jax.ShapeDtypeStruct((_NC, _NPS, 16), jnp.float32),
    mesh=plsc.VectorSubcoreMesh(core_axis_name="c", subcore_axis_name="s"),
    compiler_params=pltpu.CompilerParams(needs_layout_passes=False, use_tc_tiling_on_sc=False),
    scratch_types=[
        pltpu.VMEM((_CB,), jnp.int32),
        pltpu.VMEM((_CB,), jnp.int32),
        pltpu.VMEM((_CB, 16), jnp.float32),
        pltpu.VMEM_SHARED((_NPS, 16), jnp.float32),
        pltpu.SemaphoreType.DMA,
    ],
)


_CE = 800
_CV = 2000


def _gat_edge_body(as2_hbm, ad2_hbm, sv_hbm, dv_hbm, ex_hbm, den_hbm,
                   svb, dvb, asb, adb, exb, acc):
    c = lax.axis_index("c")
    s = lax.axis_index("s")
    epb = _E // _NS
    base = s * epb
    a_s = as2_hbm.at[c]
    a_d = ad2_hbm.at[c]
    _zero_rows128(acc, _ROWS)

    def chunk(k, carry):
        off = base + k * _CE
        pltpu.sync_copy(sv_hbm.at[pl.ds(off, _CE)], svb)
        pltpu.sync_copy(dv_hbm.at[pl.ds(off, _CE)], dvb)
        pltpu.sync_copy(a_s.at[svb], asb)
        pltpu.sync_copy(a_d.at[dvb], adb)

        def inner(j, c2):
            av = asb[pl.ds(j * 16, 16)] + adb[pl.ds(j * 16, 16)]
            av = jnp.where(av > 0, av, 0.2 * av)
            ex = jnp.exp(av)
            exb[pl.ds(j * 16, 16)] = ex
            dvv = dvb[pl.ds(j * 16, 16)]
            plsc.addupdate_scatter(acc, [dvv >> 7, dvv & 127], ex)
            return c2

        lax.fori_loop(0, _CE // 16, inner, 0, unroll=4)
        pltpu.sync_copy(exb, ex_hbm.at[c, pl.ds(off, _CE)])
        return carry

    lax.fori_loop(0, epb // _CE, chunk, 0)
    pltpu.sync_copy(acc, den_hbm.at[c, s])


_gat_edge_kernel = pl.kernel(
    _gat_edge_body,
    out_type=(jax.ShapeDtypeStruct((_NC, _E), jnp.float32),
              jax.ShapeDtypeStruct((_NC, _NS, _ROWS, 128), jnp.float32)),
    mesh=plsc.VectorSubcoreMesh(core_axis_name="c", subcore_axis_name="s"),
    compiler_params=pltpu.CompilerParams(needs_layout_passes=False, use_tc_tiling_on_sc=False),
    scratch_types=[
        pltpu.VMEM((_CE,), jnp.int32),
        pltpu.VMEM((_CE,), jnp.int32),
        pltpu.VMEM((_CE,), jnp.float32),
        pltpu.VMEM((_CE,), jnp.float32),
        pltpu.VMEM((_CE,), jnp.float32),
        pltpu.VMEM((_ROWS, 128), jnp.float32),
    ],
)


def _gat_msum_body(h2q_hbm, ex_hbm, sv_hbm, dv_hbm, zero_hbm, out_hbm,
                   svb, dvb, exb, rows, spacc, sem):
    c = lax.axis_index("c")
    s = lax.axis_index("s")
    epb = _E // _NS
    base = s * epb
    ex = ex_hbm.at[c]

    for p in range(2):
        q = 2 * c + p
        tab = h2q_hbm.at[q]
        pltpu.sync_copy(zero_hbm.at[pl.ds(s * 6256, 6256)], spacc.at[pl.ds(s * 6256, 6256)])
        plsc.subcore_barrier()

        def chunk(k, carry):
            off = base + k * _CE
            pltpu.sync_copy(sv_hbm.at[pl.ds(off, _CE)], svb)
            pltpu.sync_copy(dv_hbm.at[pl.ds(off, _CE)], dvb)
            pltpu.sync_copy(ex.at[pl.ds(off, _CE)], exb)
            pltpu.async_copy(tab.at[svb], rows, sem).wait()

            def scale(j, c2):
                exv = exb[pl.ds(j * 16, 16)]
                for t in range(16):
                    rows[j * 16 + t] = rows[j * 16 + t] * exv[t]
                return c2

            lax.fori_loop(0, _CE // 16, scale, 0, unroll=2)
            pltpu.sync_copy(rows, spacc.at[dvb], add=True)
            return carry

        lax.fori_loop(0, epb // _CE, chunk, 0)
        plsc.subcore_barrier()
        pltpu.sync_copy(spacc.at[pl.ds(s * 6256, 6256)], out_hbm.at[q, pl.ds(s * 6256, 6256)])
        plsc.subcore_barrier()


_gat_msum_kernel = pl.kernel(
    _gat_msum_body,
    out_type=jax.ShapeDtypeStruct((2 * _NC, _NPS, 16), jnp.float32),
    mesh=plsc.VectorSubcoreMesh(core_axis_name="c", subcore_axis_name="s"),
    compiler_params=pltpu.CompilerParams(needs_layout_passes=False, use_tc_tiling_on_sc=False),
    scratch_types=[
        pltpu.VMEM((_CE,), jnp.int32),
        pltpu.VMEM((_CE,), jnp.int32),
        pltpu.VMEM((_CE,), jnp.float32),
        pltpu.VMEM((_CE, 16), jnp.float32),
        pltpu.VMEM_SHARED((_NPS, 16), jnp.float32),
        pltpu.SemaphoreType.DMA,
    ],
)


def _spmv_body(vals_hbm, sv_hbm, dv_hbm, out_hbm, svb, dvb, vb, acc):
    c = lax.axis_index("c")
    s = lax.axis_index("s")
    epb = _E // _NW
    base = (c * _NS + s) * epb
    _zero_rows128(acc, _ROWS)

    def chunk(k, carry):
        off = base + k * _CV
        pltpu.sync_copy(sv_hbm.at[pl.ds(off, _CV)], svb)
        pltpu.sync_copy(dv_hbm.at[pl.ds(off, _CV)], dvb)
        pltpu.sync_copy(vals_hbm.at[svb], vb)

        def inner(j, c2):
            vv = vb[pl.ds(j * 16, 16)]
            dvv = dvb[pl.ds(j * 16, 16)]
            plsc.addupdate_scatter(acc, [dvv >> 7, dvv & 127], vv)
            return c2

        lax.fori_loop(0, _CV // 16, inner, 0, unroll=4)
        return carry

    lax.fori_loop(0, epb // _CV, chunk, 0)
    pltpu.sync_copy(acc, out_hbm.at[c, s])


_spmv_kernel = pl.kernel(
    _spmv_body,
    out_type=jax.ShapeDtypeStruct((_NC, _NS, _ROWS, 128), jnp.float32),
    mesh=plsc.VectorSubcoreMesh(core_axis_name="c", subcore_axis_name="s"),
    compiler_params=pltpu.CompilerParams(needs_layout_passes=False, use_tc_tiling_on_sc=False),
    scratch_types=[
        pltpu.VMEM((_CV,), jnp.int32),
        pltpu.VMEM((_CV,), jnp.int32),
        pltpu.VMEM((_CV,), jnp.float32),
        pltpu.VMEM((_ROWS, 128), jnp.float32),
    ],
)


_count_kernel = pl.kernel(
    _count_body,
    out_type=jax.ShapeDtypeStruct((_NC, _NS, _ROWS, 128), jnp.float32),
    mesh=plsc.VectorSubcoreMesh(core_axis_name="c", subcore_axis_name="s"),
    compiler_params=pltpu.CompilerParams(needs_layout_passes=False),
    scratch_types=[
        pltpu.VMEM((_ROWS, 128), jnp.float32),
        pltpu.VMEM((2000,), jnp.int32),
    ],
)


def _relu_kernel(x_ref, o_ref):
    o_ref[...] = jnp.maximum(x_ref[...], 0.0)


def _pallas_relu(x):
    rows = x.shape[0]
    blk = 10000
    return pl.pallas_call(
        _relu_kernel,
        grid=(rows // blk,),
        in_specs=[pl.BlockSpec((blk, x.shape[1]), lambda i: (i, 0))],
        out_specs=pl.BlockSpec((blk, x.shape[1]), lambda i: (i, 0)),
        out_shape=jax.ShapeDtypeStruct(x.shape, x.dtype),
    )(x)


def kernel(x, edge_index, W1, b1, Wg, att_src, att_dst, bg, W2, b2):
    n = x.shape[0]
    src = edge_index[0]
    dst = edge_index[1]

    degp = _count_kernel(dst).reshape(_NW, _NPAD)[:, :n]
    deg = degp.sum(0) + 1.0
    dis = deg ** -0.5

    h1 = x @ W1.T
    hs1 = h1 * dis[:, None]
    hs1h = jnp.stack([hs1[:, :16], hs1[:, 16:]])
    zpad = jnp.zeros((_NPS, 16), jnp.float32)
    aggp = _spmm16_kernel(hs1h, src, dst, zpad)
    agg1 = jnp.concatenate([aggp[0, :n], aggp[1, :n]], axis=1)
    out1 = dis[:, None] * (agg1 + hs1) + b1
    r1 = _pallas_relu(out1)

    h2f = r1 @ Wg.T
    h2 = h2f.reshape(n, _HEADS, _HID)
    a_s = (h2 * att_src).sum(-1)
    a_d = (h2 * att_dst).sum(-1)
    exh, denp = _gat_edge_kernel(a_s.T, a_d.T, src, dst)
    den = denp.reshape(_NC, _NS, _NPAD)[:, :, :n].sum(1).T
    h2q = jnp.stack([h2f[:, 16 * q:16 * (q + 1)] for q in range(4)])
    msump = _gat_msum_kernel(h2q, exh, src, dst, zpad)
    msum = jnp.concatenate([msump[q, :n] for q in range(4)], axis=1)
    msum = msum.reshape(n, _HEADS, _HID)
    a_self = a_s + a_d
    ex_self = jnp.exp(jnp.where(a_self > 0, a_self, 0.2 * a_self))
    den = den + ex_self
    msum = msum + h2 * ex_self[:, :, None]
    out2 = (msum / (den[:, :, None] + 1e-16)).mean(axis=1) + bg
    r2 = _pallas_relu(out2)

    h3 = (r2 @ W2.T)[:, 0]
    hs3 = h3 * dis
    aggv = _spmv_kernel(hs3, src, dst)
    agg3 = aggv.reshape(_NW, _NPAD)[:, :n].sum(0)
    out3 = dis * (agg3 + hs3) + b2[0]
    return out3
